# baseline (speedup 1.0000x reference)
import functools

import jax
import jax.numpy as jnp
from jax import lax
from jax.experimental import pallas as pl
from jax.experimental.pallas import tpu as pltpu
from jax.experimental.pallas import tpu_sc as plsc

_B = 16384
_AUDIO = 128
_D = 256
_AD = 32
_NC = 2
_NS = 16
_NW = _NC * _NS
_BPW = _B // _NW
_CH = 128
_NCHUNK = _BPW // _CH


def _sc_gather_item(item_idx, item_tab, nb):
    bpw = nb // _NW
    nchunk = bpw // _CH
    mesh = plsc.VectorSubcoreMesh(core_axis_name="c", subcore_axis_name="s")

    @functools.partial(
        pl.kernel,
        mesh=mesh,
        out_type=jax.ShapeDtypeStruct((nb, _D), jnp.float32),
        scratch_types=[
            pltpu.VMEM((nchunk, _CH), jnp.int32),
            pltpu.VMEM((_CH, _D), jnp.float32),
            pltpu.VMEM((_CH, _D), jnp.float32),
            pltpu.SemaphoreType.DMA,
            pltpu.SemaphoreType.DMA,
            pltpu.SemaphoreType.DMA,
            pltpu.SemaphoreType.DMA,
        ],
    )
    def k(item_idx_h, item_tab_h, out_item, iidx, ibuf0, ibuf1, g0, g1, o0,
          o1):
        wid = lax.axis_index("s") * _NC + lax.axis_index("c")
        base = wid * bpw
        pltpu.sync_copy(item_idx_h.at[wid], iidx)

        ibufs = (ibuf0, ibuf1)
        gsems = (g0, g1)
        osems = (o0, o1)
        gcp = [None] * nchunk
        ocp = [None] * nchunk
        gcp[0] = pltpu.async_copy(item_tab_h.at[iidx.at[0]], ibufs[0], gsems[0])
        for c in range(nchunk):
            s = c % 2
            if c + 1 < nchunk:
                if c - 1 >= 0:
                    ocp[c - 1].wait()
                gcp[c + 1] = pltpu.async_copy(
                    item_tab_h.at[iidx.at[c + 1]], ibufs[(c + 1) % 2],
                    gsems[(c + 1) % 2])
            gcp[c].wait()
            ocp[c] = pltpu.async_copy(
                ibufs[s], out_item.at[pl.ds(base + c * _CH, _CH)], osems[s])

        for cp in ocp[-2:]:
            cp.wait()

    return k(item_idx, item_tab)


def _ln(x, g, b):
    m = jnp.mean(x, axis=-1, keepdims=True)
    v = jnp.mean((x - m) ** 2, axis=-1, keepdims=True)
    return (x - m) / jnp.sqrt(v + 1e-5) * g + b


_BM = 4096


def _ln_cols(x, g, b):
    m = jnp.mean(x, axis=0, keepdims=True)
    v = jnp.mean((x - m) ** 2, axis=0, keepdims=True)
    return (x - m) / jnp.sqrt(v + 1e-5) * g + b


def _tc_body(prev, ie, idr, art, alt, wa, w1d, w1i, w1ab, w2, p256, p32t,
             out):
    f32 = jnp.float32
    bf16 = jnp.bfloat16
    dense = jnp.dot(ie[...], wa[...], preferred_element_type=f32) + p256[0:1, :]
    dense = jnp.maximum(_ln(dense, p256[1:2, :], p256[2:3, :]), 0.0)
    idv = _ln(idr[...], p256[3:4, :], p256[4:5, :])
    arv = _ln_cols(art[...], p32t[:, 0:1], p32t[:, 1:2])
    alv = _ln_cols(alt[...], p32t[:, 2:3], p32t[:, 3:4])
    ab_t = jnp.concatenate([arv, alv], axis=0)
    h = (jnp.dot(dense.astype(bf16), w1d[...].astype(bf16),
                 preferred_element_type=f32)
         + jnp.dot(idv.astype(bf16), w1i[...].astype(bf16),
                   preferred_element_type=f32)
         + lax.dot_general(ab_t.astype(bf16), w1ab[...].astype(bf16),
                           (((0,), (0,)), ((), ())),
                           preferred_element_type=f32)
         + p256[5:6, :])
    h = jnp.maximum(_ln(h, p256[6:7, :], p256[7:8, :]), 0.0)
    h = jnp.dot(h.astype(bf16), w2[...].astype(bf16),
                preferred_element_type=f32) + p256[8:9, :]
    h = jnp.maximum(_ln(h, p256[9:10, :], p256[10:11, :]), 0.0)
    n = jnp.sqrt(jnp.sum(h * h, axis=-1, keepdims=True))
    out[...] = h / jnp.maximum(n, 1e-12)


def _tc_forward(prev, off, nb, item_embed, id_rows, art_t, alb_t, Wa, W1d,
                W1i, W1ab, W2, P256, P32T):
    grid = (nb // _BM,)
    ob = off // _BM
    specs = [
        pl.BlockSpec((_BM, _AUDIO), lambda i: (ob + i, 0)),
        pl.BlockSpec((_BM, _D), lambda i: (ob + i, 0)),
        pl.BlockSpec((_AD, _BM), lambda i: (0, i)),
        pl.BlockSpec((_AD, _BM), lambda i: (0, i)),
        pl.BlockSpec((_AUDIO, _D), lambda i: (0, 0)),
        pl.BlockSpec((_D, _D), lambda i: (0, 0)),
        pl.BlockSpec((_D, _D), lambda i: (0, 0)),
        pl.BlockSpec((2 * _AD, _D), lambda i: (0, 0)),
        pl.BlockSpec((_D, _D), lambda i: (0, 0)),
        pl.BlockSpec((16, _D), lambda i: (0, 0)),
        pl.BlockSpec((_AD, 8), lambda i: (0, 0)),
    ]
    args = [item_embed, id_rows, art_t, alb_t, Wa, W1d, W1i, W1ab, W2, P256,
            P32T]
    body = _tc_body
    aliases = {}
    if prev is not None:
        specs = [pl.BlockSpec(memory_space=pl.ANY)] + specs
        args = [prev] + args
        aliases = {0: 0}
    else:
        def body(*refs):
            return _tc_body(None, *refs)
    return pl.pallas_call(
        body,
        grid=grid,
        in_specs=specs,
        out_specs=pl.BlockSpec((_BM, _D), lambda i: (ob + i, 0)),
        out_shape=jax.ShapeDtypeStruct((_B, _D), jnp.float32),
        input_output_aliases=aliases,
    )(*args)


def kernel(item_embed, target_item_id, item_artist_id, item_album_id,
           item_table, artist_table, album_table, Wa, ba, g_audio, b_audio,
           g_id, b_id, g_art, b_art, g_alb, b_alb, W1, b1, g1, be1, W2, b2,
           g2, be2):
    W1d = W1[:_D]
    W1i = W1[_D:2 * _D]
    W1ab = W1[2 * _D:]
    P256 = jnp.concatenate([
        jnp.stack([ba, g_audio, b_audio, g_id, b_id, b1, g1, be1, b2, g2,
                   be2]),
        jnp.zeros((5, _D), jnp.float32),
    ])
    P32T = jnp.concatenate([
        jnp.stack([g_art, b_art, g_alb, b_alb], axis=1),
        jnp.zeros((_AD, 4), jnp.float32),
    ], axis=1)

    def zdep(x):
        return (x.reshape(-1)[0] * 0.0).astype(jnp.int32)

    nsplit = 2
    nb = _B // nsplit
    tid = target_item_id.astype(jnp.int32)
    aid = item_artist_id.astype(jnp.int32)
    lid = item_album_id.astype(jnp.int32)
    out = None
    iidx = tid.reshape(_NW, _B // _NW // _CH, _CH)
    id_rows = _sc_gather_item(iidx, item_table, _B)
    gdep = zdep(id_rows)
    for s in range(nsplit):
        sl = slice(s * nb, (s + 1) * nb)
        art_t = artist_table.at[aid[sl] + gdep].get(
            mode="promise_in_bounds").T
        alb_t = album_table.at[lid[sl] + gdep].get(
            mode="promise_in_bounds").T
        gdep = zdep(art_t) + zdep(alb_t)
        out = _tc_forward(out, s * nb, nb, item_embed, id_rows, art_t, alb_t,
                          Wa, W1d, W1i, W1ab, W2, P256, P32T)
    return out


# speedup vs baseline: 1.0452x; 1.0452x over previous
import functools

import jax
import jax.numpy as jnp
from jax import lax
from jax.experimental import pallas as pl
from jax.experimental.pallas import tpu as pltpu
from jax.experimental.pallas import tpu_sc as plsc

_B = 16384
_AUDIO = 128
_D = 256
_AD = 32
_NC = 2
_NS = 16
_NW = _NC * _NS
_BPW = _B // _NW
_CH = 128
_NCHUNK = _BPW // _CH


def _sc_gather_item(item_idx, item_tab, nb):
    bpw = nb // _NW
    nchunk = bpw // _CH
    mesh = plsc.VectorSubcoreMesh(core_axis_name="c", subcore_axis_name="s")

    @functools.partial(
        pl.kernel,
        mesh=mesh,
        out_type=jax.ShapeDtypeStruct((nb, _D), jnp.float32),
        scratch_types=[
            pltpu.VMEM((nchunk, _CH), jnp.int32),
            pltpu.VMEM((_CH, _D), jnp.float32),
            pltpu.VMEM((_CH, _D), jnp.float32),
            pltpu.SemaphoreType.DMA,
            pltpu.SemaphoreType.DMA,
            pltpu.SemaphoreType.DMA,
            pltpu.SemaphoreType.DMA,
        ],
    )
    def k(item_idx_h, item_tab_h, out_item, iidx, ibuf0, ibuf1, g0, g1, o0,
          o1):
        wid = lax.axis_index("s") * _NC + lax.axis_index("c")
        base = wid * bpw
        pltpu.sync_copy(item_idx_h.at[wid], iidx)

        ibufs = (ibuf0, ibuf1)
        gsems = (g0, g1)
        osems = (o0, o1)
        gcp = [None] * nchunk
        ocp = [None] * nchunk
        gcp[0] = pltpu.async_copy(item_tab_h.at[iidx.at[0]], ibufs[0], gsems[0])
        for c in range(nchunk):
            s = c % 2
            if c + 1 < nchunk:
                if c - 1 >= 0:
                    ocp[c - 1].wait()
                gcp[c + 1] = pltpu.async_copy(
                    item_tab_h.at[iidx.at[c + 1]], ibufs[(c + 1) % 2],
                    gsems[(c + 1) % 2])
            gcp[c].wait()
            ocp[c] = pltpu.async_copy(
                ibufs[s], out_item.at[pl.ds(base + c * _CH, _CH)], osems[s])

        for cp in ocp[-2:]:
            cp.wait()

    return k(item_idx, item_tab)


def _ln(x, g, b):
    m = jnp.mean(x, axis=-1, keepdims=True)
    v = jnp.mean((x - m) ** 2, axis=-1, keepdims=True)
    return (x - m) / jnp.sqrt(v + 1e-5) * g + b


_BM = 2048


def _ln_cols(x, g, b):
    m = jnp.mean(x, axis=0, keepdims=True)
    v = jnp.mean((x - m) ** 2, axis=0, keepdims=True)
    return (x - m) / jnp.sqrt(v + 1e-5) * g + b


def _tc_body(prev, ie, idr, art, alt, wa, w1d, w1i, w1ab, w2, p256, p32t,
             out):
    f32 = jnp.float32
    bf16 = jnp.bfloat16
    dense = jnp.dot(ie[...], wa[...], preferred_element_type=f32) + p256[0:1, :]
    dense = jnp.maximum(_ln(dense, p256[1:2, :], p256[2:3, :]), 0.0)
    idv = _ln(idr[...], p256[3:4, :], p256[4:5, :])
    arv = _ln_cols(art[...], p32t[:, 0:1], p32t[:, 1:2])
    alv = _ln_cols(alt[...], p32t[:, 2:3], p32t[:, 3:4])
    ab_t = jnp.concatenate([arv, alv], axis=0)
    h = (jnp.dot(dense.astype(bf16), w1d[...].astype(bf16),
                 preferred_element_type=f32)
         + jnp.dot(idv.astype(bf16), w1i[...].astype(bf16),
                   preferred_element_type=f32)
         + lax.dot_general(ab_t.astype(bf16), w1ab[...].astype(bf16),
                           (((0,), (0,)), ((), ())),
                           preferred_element_type=f32)
         + p256[5:6, :])
    h = jnp.maximum(_ln(h, p256[6:7, :], p256[7:8, :]), 0.0)
    h = jnp.dot(h.astype(bf16), w2[...].astype(bf16),
                preferred_element_type=f32) + p256[8:9, :]
    h = jnp.maximum(_ln(h, p256[9:10, :], p256[10:11, :]), 0.0)
    n = jnp.sqrt(jnp.sum(h * h, axis=-1, keepdims=True))
    out[...] = h / jnp.maximum(n, 1e-12)


def _tc_forward(prev, off, nb, item_embed, id_rows, art_t, alb_t, Wa, W1d,
                W1i, W1ab, W2, P256, P32T):
    grid = (nb // _BM,)
    ob = off // _BM
    specs = [
        pl.BlockSpec((_BM, _AUDIO), lambda i: (ob + i, 0)),
        pl.BlockSpec((_BM, _D), lambda i: (ob + i, 0)),
        pl.BlockSpec((_AD, _BM), lambda i: (0, i)),
        pl.BlockSpec((_AD, _BM), lambda i: (0, i)),
        pl.BlockSpec((_AUDIO, _D), lambda i: (0, 0)),
        pl.BlockSpec((_D, _D), lambda i: (0, 0)),
        pl.BlockSpec((_D, _D), lambda i: (0, 0)),
        pl.BlockSpec((2 * _AD, _D), lambda i: (0, 0)),
        pl.BlockSpec((_D, _D), lambda i: (0, 0)),
        pl.BlockSpec((16, _D), lambda i: (0, 0)),
        pl.BlockSpec((_AD, 8), lambda i: (0, 0)),
    ]
    args = [item_embed, id_rows, art_t, alb_t, Wa, W1d, W1i, W1ab, W2, P256,
            P32T]
    body = _tc_body
    aliases = {}
    if prev is not None:
        specs = [pl.BlockSpec(memory_space=pl.ANY)] + specs
        args = [prev] + args
        aliases = {0: 0}
    else:
        def body(*refs):
            return _tc_body(None, *refs)
    return pl.pallas_call(
        body,
        grid=grid,
        in_specs=specs,
        out_specs=pl.BlockSpec((_BM, _D), lambda i: (ob + i, 0)),
        out_shape=jax.ShapeDtypeStruct((_B, _D), jnp.float32),
        input_output_aliases=aliases,
    )(*args)


def kernel(item_embed, target_item_id, item_artist_id, item_album_id,
           item_table, artist_table, album_table, Wa, ba, g_audio, b_audio,
           g_id, b_id, g_art, b_art, g_alb, b_alb, W1, b1, g1, be1, W2, b2,
           g2, be2):
    W1d = W1[:_D]
    W1i = W1[_D:2 * _D]
    W1ab = W1[2 * _D:]
    P256 = jnp.concatenate([
        jnp.stack([ba, g_audio, b_audio, g_id, b_id, b1, g1, be1, b2, g2,
                   be2]),
        jnp.zeros((5, _D), jnp.float32),
    ])
    P32T = jnp.concatenate([
        jnp.stack([g_art, b_art, g_alb, b_alb], axis=1),
        jnp.zeros((_AD, 4), jnp.float32),
    ], axis=1)

    def zdep(x):
        return (x.reshape(-1)[0] * 0.0).astype(jnp.int32)

    nsplit = 2
    nb = _B // nsplit
    tid = target_item_id.astype(jnp.int32)
    aid = item_artist_id.astype(jnp.int32)
    lid = item_album_id.astype(jnp.int32)
    out = None
    iidx = tid.reshape(_NW, _B // _NW // _CH, _CH)
    id_rows = _sc_gather_item(iidx, item_table, _B)
    gdep = zdep(id_rows)
    for s in range(nsplit):
        sl = slice(s * nb, (s + 1) * nb)
        art_t = artist_table.at[aid[sl] + gdep].get(
            mode="promise_in_bounds").T
        alb_t = album_table.at[lid[sl] + gdep].get(
            mode="promise_in_bounds").T
        gdep = zdep(art_t) + zdep(alb_t)
        out = _tc_forward(out, s * nb, nb, item_embed, id_rows, art_t, alb_t,
                          Wa, W1d, W1i, W1ab, W2, P256, P32T)
    return out
